# baseline (speedup 1.0000x reference)
import functools
import math

import jax
import jax.numpy as jnp
from jax import lax
from jax.experimental import pallas as pl
from jax.experimental.pallas import tpu as pltpu

B_, N_, E_, D_ = 4, 2048, 16384, 64
NR, NL, TAU, M_ = 500, 3, 0.1, 20
BE = B_ * E_
BN = B_ * N_
EC = 2048
NEC = BE // EC
CPB = E_ // EC


def _pre_body(scores_ref, ecm_ref, hr_ref, qr_ref, confB_ref, confW_ref,
              confb_ref, rel_ref, betaW_ref, betab_ref, conf_ref, gate_ref):
    b = pl.program_id(0) // CPB
    s = scores_ref[...]
    m = ecm_ref[...]
    s3 = s * m
    xp = (2.0 * math.pi) * s3 * confB_ref[...]
    cs = jnp.concatenate([jnp.cos(xp), jnp.sin(xp)], axis=1)
    conf_ref[...] = cs @ confW_ref[...] + confb_ref[...]
    rtb = rel_ref[...] @ betaW_ref[...]
    qr = qr_ref[...]
    i500 = lax.broadcasted_iota(jnp.int32, (B_, NR), 1)
    qoh = (qr == i500).astype(jnp.float32)
    rqbw = qoh @ rtb
    i4 = lax.broadcasted_iota(jnp.int32, (B_, 1), 0)
    rqbw_b = jnp.sum(jnp.where(i4 == b, rqbw, 0.0), axis=0, keepdims=True)
    beta = jax.nn.sigmoid(hr_ref[...] @ betaW_ref[...] + rqbw_b + betab_ref[...])
    gate = m * jax.nn.sigmoid((s - beta) / TAU) + (1.0 - m) * 0.5
    gate_ref[...] = gate


def _precompute(scores_f, ecm_f, h_r, query_rels, conf_B, conf_W, conf_b,
                rel_table, beta_W, beta_b):
    full = lambda shape: pl.BlockSpec(shape, lambda i: (0, 0))
    chunk = lambda w: pl.BlockSpec((EC, w), lambda i: (i, 0))
    return pl.pallas_call(
        _pre_body,
        grid=(NEC,),
        in_specs=[chunk(1), chunk(1), chunk(D_), full((B_, 1)),
                  full((1, D_ // 2)), full((D_, D_)), full((1, D_)),
                  full((NR, D_)), full((D_, 1)), full((1, 1))],
        out_specs=[chunk(D_), chunk(1)],
        out_shape=[jax.ShapeDtypeStruct((BE, D_), jnp.float32),
                   jax.ShapeDtypeStruct((BE, 1), jnp.float32)],
    )(scores_f, ecm_f, h_r, query_rels, conf_B, conf_W, conf_b,
      rel_table, beta_W, beta_b)


def _lre_msg_body(hs_ref, hr_ref, cf_ref, gate_ref, src0_ref, Wc_ref,
                  csum_ref, bk_ref, wm_ref):
    hs = hs_ref[...]
    hr = hr_ref[...]
    x = jnp.concatenate([hs * hr, hs, hr, cf_ref[...]], axis=1)
    raw = x @ Wc_ref[...] + src0_ref[...] * csum_ref[...] + bk_ref[...]
    wm_ref[...] = gate_ref[...] * jnp.maximum(raw, 0.0)


def _lre_msg(h_src, h_r, conf, gate, src0, Wc, csum3, bk):
    full = lambda shape: pl.BlockSpec(shape, lambda i: (0, 0))
    chunk = lambda w: pl.BlockSpec((EC, w), lambda i: (i, 0))
    return pl.pallas_call(
        _lre_msg_body,
        grid=(NEC,),
        in_specs=[chunk(D_), chunk(D_), chunk(D_), chunk(1), chunk(1),
                  full((4 * D_, D_)), full((1, D_)), full((1, D_))],
        out_specs=chunk(D_),
        out_shape=jax.ShapeDtypeStruct((BE, D_), jnp.float32),
    )(h_src, h_r, conf, gate, src0, Wc, csum3, bk)


def _sfe_msg_body(hs_ref, hr_ref, d16_ref, cf_ref, Wc_ref, t10_ref, bk_ref,
                  wm_ref):
    hs = hs_ref[...]
    hr = hr_ref[...]
    x = jnp.concatenate([hs * hr, hs, hr, cf_ref[...]], axis=1)
    dval = d16_ref[...][:, 0:1]
    i10 = lax.broadcasted_iota(jnp.int32, (EC, 10), 1).astype(jnp.float32)
    oneh = (dval == i10).astype(jnp.float32)
    wm_ref[...] = jnp.maximum(
        x @ Wc_ref[...] + oneh @ t10_ref[...] + bk_ref[...], 0.0)


def _sfe_msg(h_src, h_r, d16, conf, Wc, tbl10, bk):
    full = lambda shape: pl.BlockSpec(shape, lambda i: (0, 0))
    chunk = lambda w: pl.BlockSpec((EC, w), lambda i: (i, 0))
    return pl.pallas_call(
        _sfe_msg_body,
        grid=(NEC,),
        in_specs=[chunk(D_), chunk(D_), chunk(16), chunk(D_),
                  full((4 * D_, D_)), full((10, D_)), full((1, D_))],
        out_specs=chunk(D_),
        out_shape=jax.ShapeDtypeStruct((BE, D_), jnp.float32),
    )(h_src, h_r, d16, conf, Wc, tbl10, bk)


def _ln_rows(x, g, b):
    m = jnp.mean(x, axis=1, keepdims=True)
    v = jnp.mean((x - m) ** 2, axis=1, keepdims=True)
    return (x - m) / jnp.sqrt(v + 1e-5) * g + b


def _lre_upd_body(p0_ref, h_ref, W_ref, b_ref, g_ref, lb_ref, o_ref):
    aggr = p0_ref[...]
    o_ref[...] = _ln_rows(h_ref[...] + aggr @ W_ref[...] + b_ref[...],
                          g_ref[...], lb_ref[...])


def _lre_upd(p0, h, W, b, g, lb):
    full = lambda shape: pl.BlockSpec(shape, lambda: (0, 0))
    return pl.pallas_call(
        _lre_upd_body,
        in_specs=[full((BN, D_)), full((BN, D_)),
                  full((D_, D_)), full((1, D_)), full((1, D_)), full((1, D_))],
        out_specs=full((BN, D_)),
        out_shape=jax.ShapeDtypeStruct((BN, D_), jnp.float32),
    )(p0, h, W, b, g, lb)


def _sfe_upd_body(p0_ref, h_ref, W_ref, b_ref, o_ref):
    o_ref[...] = h_ref[...] + p0_ref[...] @ W_ref[...] + b_ref[...]


def _sfe_upd(p0, h, W, b):
    full = lambda shape: pl.BlockSpec(shape, lambda: (0, 0))
    return pl.pallas_call(
        _sfe_upd_body,
        in_specs=[full((BN, D_)), full((BN, D_)),
                  full((D_, D_)), full((1, D_))],
        out_specs=full((BN, D_)),
        out_shape=jax.ShapeDtypeStruct((BN, D_), jnp.float32),
    )(p0, h, W, b)


def _add_body(a_ref, b_ref, o_ref):
    o_ref[...] = a_ref[...] + b_ref[...]


def _add2(a, b):
    full = pl.BlockSpec((BN, D_), lambda: (0, 0))
    return pl.pallas_call(
        _add_body,
        in_specs=[full, full],
        out_specs=full,
        out_shape=jax.ShapeDtypeStruct((BN, D_), jnp.float32),
    )(a, b)


def _finale_body(ctx_ref, h2f_ref, qr_ref, rel_ref,
                 attW1_ref, attW2_ref, attb_ref, Wq_ref, bq_ref, Wk_ref,
                 bk_ref, Wv_ref, bv_ref, g_ref, lb_ref, o_ref):
    b = pl.program_id(0)
    h2f = h2f_ref[...]
    t_state = h2f[0:1, :]
    qr = qr_ref[...]
    i4 = lax.broadcasted_iota(jnp.int32, (B_, 1), 0)
    qr_b = jnp.sum(jnp.where(i4 == b, qr, 0), axis=0, keepdims=True)
    i500c = lax.broadcasted_iota(jnp.int32, (1, NR), 1)
    qoh = (i500c == qr_b).astype(jnp.float32)
    rq = qoh @ rel_ref[...]
    att = h2f @ attW1_ref[...] + (rq @ attW2_ref[...] + attb_ref[...])
    att = jnp.where(att >= 0.0, att, 0.01 * att)
    amax = jnp.max(att, axis=0, keepdims=True)
    ex = jnp.exp(att - amax)
    alpha = ex / jnp.sum(ex, axis=0, keepdims=True)
    iota = lax.broadcasted_iota(jnp.int32, (N_, 1), 0)
    acur = alpha
    rows = []
    for _ in range(M_):
        v = jnp.max(acur, axis=0, keepdims=True)
        eq = acur == v
        fidx = jnp.min(jnp.where(eq, iota, N_), axis=0, keepdims=True)
        sel = iota == fidx
        ohf = sel.astype(jnp.float32)
        rows.append(jnp.sum(ohf * h2f, axis=0, keepdims=True) * v)
        acur = jnp.where(sel, -1.0, acur)
    ctx = ctx_ref[...].reshape(8, D_)[0:NL, :]
    x = jnp.concatenate([ctx] + rows, axis=0)
    Nt = NL + M_
    hd = lax.broadcasted_iota(jnp.int32, (D_, D_), 0) // 16
    hD = lax.broadcasted_iota(jnp.int32, (D_, D_), 1) // 16
    blockones = (hd == hD).astype(jnp.float32)
    q = x @ Wq_ref[...] + bq_ref[...]
    k_ = x @ Wk_ref[...] + bk_ref[...]
    v_ = x @ Wv_ref[...] + bv_ref[...]

    def nrmh(t):
        ssum = (t * t) @ blockones
        return t / jnp.maximum(jnp.sqrt(ssum), 1e-12)

    q = nrmh(q)
    k_ = nrmh(k_)
    KtV = lax.dot_general(k_, v_, (((0,), (0,)), ((), ())))
    kvs = KtV * blockones
    vsum = jnp.sum(v_, axis=0, keepdims=True)
    ksum = jnp.sum(k_, axis=0, keepdims=True)
    num = q @ kvs + vsum + v_ * float(Nt)
    den = (q * ksum) @ blockones + float(2 * Nt)
    out = num / den
    y = _ln_rows(x + out, g_ref[...], lb_ref[...])
    res = jnp.mean(y, axis=0, keepdims=True) + t_state
    o_ref[...] = jnp.concatenate(
        [res, jnp.zeros((7, D_), jnp.float32)], axis=0).reshape(1, 8, D_)


def _finale(ctx_all, h2f, query_rels, rel_table, attW1, attW2, attb,
            Wq, bq, Wk, bk, Wv, bv, g, lb):
    full = lambda shape: pl.BlockSpec(shape, lambda b: tuple(0 for _ in shape))
    return pl.pallas_call(
        _finale_body,
        grid=(B_,),
        in_specs=[pl.BlockSpec((1, 8, D_), lambda b: (b, 0, 0)),
                  pl.BlockSpec((N_, D_), lambda b: (b, 0)),
                  full((B_, 1)), full((NR, D_)),
                  full((D_, 1)), full((D_, 1)), full((1, 1)),
                  full((D_, D_)), full((1, D_)), full((D_, D_)), full((1, D_)),
                  full((D_, D_)), full((1, D_)), full((1, D_)), full((1, D_))],
        out_specs=pl.BlockSpec((1, 8, D_), lambda b: (b, 0, 0)),
        out_shape=jax.ShapeDtypeStruct((B_, 8, D_), jnp.float32),
    )(ctx_all, h2f, query_rels, rel_table, attW1, attW2, attb,
      Wq, bq, Wk, bk, Wv, bv, g, lb)[:, 0, :]


NC, NS = 2, 16
NW = NC * NS


def _sc_mesh():
    from jax.experimental.pallas import tpu_sc as plsc
    return plsc.VectorSubcoreMesh(core_axis_name="c", subcore_axis_name="s")


def _gather_rows(table, idx, chunk=512):
    n = idx.shape[0]
    D = table.shape[1]
    per_w = n // NW
    c = min(chunk, per_w)
    nch = per_w // c

    @functools.partial(
        pl.kernel,
        out_type=jax.ShapeDtypeStruct((n, D), jnp.float32),
        mesh=_sc_mesh(),
        scratch_types=[pltpu.VMEM((per_w,), jnp.int32),
                       pltpu.VMEM((2, c, D), jnp.float32),
                       pltpu.SemaphoreType.DMA, pltpu.SemaphoreType.DMA,
                       pltpu.SemaphoreType.DMA, pltpu.SemaphoreType.DMA],
        compiler_params=pltpu.CompilerParams(use_tc_tiling_on_sc=False),
    )
    def gk(table_hbm, idx_hbm, out_hbm, idx_v, rows_v, g0, g1, w0, w1):
        w = lax.axis_index("c") * NS + lax.axis_index("s")
        base = w * per_w
        pltpu.sync_copy(idx_hbm.at[pl.ds(base, per_w)], idx_v)
        gsem = [g0, g1]
        wsem = [w0, w1]
        gd = [None, None]
        wd = [None, None]
        gd[0] = pltpu.async_copy(table_hbm.at[idx_v.at[pl.ds(0, c)]],
                                 rows_v.at[0], gsem[0])
        for j in range(nch):
            b = j % 2
            nb = (j + 1) % 2
            if j + 1 < nch:
                if wd[nb] is not None:
                    wd[nb].wait()
                gd[nb] = pltpu.async_copy(
                    table_hbm.at[idx_v.at[pl.ds((j + 1) * c, c)]],
                    rows_v.at[nb], gsem[nb])
            gd[b].wait()
            wd[b] = pltpu.async_copy(rows_v.at[b],
                                     out_hbm.at[pl.ds(base + j * c, c)], wsem[b])
        for d in wd:
            if d is not None:
                d.wait()

    return gk(table, idx)


def _gather_consts(rel_table, rels_f, dist_table, dclip, dclip16, srcg):
    pwE = BE // NW
    pwN = BN // NW
    c = 512
    nch = pwE // c

    @functools.partial(
        pl.kernel,
        out_type=(jax.ShapeDtypeStruct((BE, D_), jnp.float32),
                  jax.ShapeDtypeStruct((BN, D_), jnp.float32),
                  jax.ShapeDtypeStruct((BE, 16), jnp.float32)),
        mesh=_sc_mesh(),
        scratch_types=[pltpu.VMEM((pwE,), jnp.int32),
                       pltpu.VMEM((pwN,), jnp.int32),
                       pltpu.VMEM((2, c, D_), jnp.float32),
                       pltpu.VMEM((pwN, D_), jnp.float32),
                       pltpu.VMEM((pwE, 16), jnp.float32),
                       pltpu.SemaphoreType.DMA, pltpu.SemaphoreType.DMA,
                       pltpu.SemaphoreType.DMA, pltpu.SemaphoreType.DMA,
                       pltpu.SemaphoreType.DMA, pltpu.SemaphoreType.DMA],
        compiler_params=pltpu.CompilerParams(use_tc_tiling_on_sc=False),
    )
    def gk(rel_hbm, rels_hbm, dtbl_hbm, dclip_hbm, d16tbl_hbm, srcg_hbm,
           hr_out, demb_out, d16_out, idxE, idxN, rows, rowsN, rows16,
           g0, g1, w0, w1, aux, aux2):
        w = lax.axis_index("c") * NS + lax.axis_index("s")
        baseE = w * pwE
        baseN = w * pwN
        pltpu.sync_copy(srcg_hbm.at[pl.ds(baseE, pwE)], idxE)
        d16g = pltpu.async_copy(d16tbl_hbm.at[idxE], rows16, aux2)
        pltpu.sync_copy(dclip_hbm.at[pl.ds(baseN, pwN)], idxN)
        dg = pltpu.async_copy(dtbl_hbm.at[idxN], rowsN, aux)
        d16g.wait()
        d16w = pltpu.async_copy(rows16, d16_out.at[pl.ds(baseE, pwE)], aux2)
        dg.wait()
        dw = pltpu.async_copy(rowsN, demb_out.at[pl.ds(baseN, pwN)], aux)
        pltpu.sync_copy(rels_hbm.at[pl.ds(baseE, pwE)], idxE)
        gsem = [g0, g1]
        wsem = [w0, w1]
        gd = [None, None]
        wd = [None, None]
        gd[0] = pltpu.async_copy(rel_hbm.at[idxE.at[pl.ds(0, c)]],
                                 rows.at[0], gsem[0])
        for j in range(nch):
            b = j % 2
            nb = (j + 1) % 2
            if j + 1 < nch:
                if wd[nb] is not None:
                    wd[nb].wait()
                gd[nb] = pltpu.async_copy(
                    rel_hbm.at[idxE.at[pl.ds((j + 1) * c, c)]],
                    rows.at[nb], gsem[nb])
            gd[b].wait()
            wd[b] = pltpu.async_copy(rows.at[b],
                                     hr_out.at[pl.ds(baseE + j * c, c)],
                                     wsem[b])
        for d in wd:
            if d is not None:
                d.wait()
        d16w.wait()
        dw.wait()

    return gk(rel_table, rels_f, dist_table, dclip, dclip16, srcg)


def _scatter_add_bn(vals, idx3, zeros_half):
    KCH = E_ * B_ // NW // 128
    HALF = BN // NC
    RPS = HALF // NS

    @functools.partial(
        pl.kernel,
        out_type=jax.ShapeDtypeStruct((BN, D_), jnp.float32),
        mesh=_sc_mesh(),
        scratch_types=[pltpu.VMEM((KCH, 128), jnp.int32),
                       pltpu.VMEM((2, 512, D_), jnp.float32),
                       pltpu.VMEM_SHARED((HALF, D_), jnp.float32),
                       pltpu.SemaphoreType.DMA, pltpu.SemaphoreType.DMA,
                       pltpu.SemaphoreType.DMA, pltpu.SemaphoreType.DMA],
        compiler_params=pltpu.CompilerParams(use_tc_tiling_on_sc=False),
    )
    def sk(vals_hbm, idx_hbm, zeros_hbm, out_hbm, idx_v, vals_v, acc,
           l0, l1, s0, s1):
        from jax.experimental.pallas import tpu_sc as plsc
        cid = lax.axis_index("c")
        sid = lax.axis_index("s")
        w = cid * NS + sid
        pltpu.sync_copy(zeros_hbm.at[pl.ds(sid * RPS, RPS)],
                        acc.at[pl.ds(sid * RPS, RPS)])
        pltpu.sync_copy(idx_hbm.at[w], idx_v)
        plsc.subcore_barrier()
        base = w * (KCH * 128)
        lsem = [l0, l1]
        ssem = [s0, s1]
        ld = [None, None]
        sd = [[], []]
        ld[0] = pltpu.async_copy(vals_hbm.at[pl.ds(base, 512)],
                                 vals_v.at[0], lsem[0])
        for j in range(4):
            b = j % 2
            nb = (j + 1) % 2
            if j + 1 < 4:
                for d in sd[nb]:
                    d.wait()
                sd[nb] = []
                ld[nb] = pltpu.async_copy(
                    vals_hbm.at[pl.ds(base + (j + 1) * 512, 512)],
                    vals_v.at[nb], lsem[nb])
            ld[b].wait()
            sd[b] = [pltpu.async_copy(vals_v.at[b].at[pl.ds(t * 128, 128)],
                                      acc.at[idx_v.at[j * 4 + t]], ssem[b],
                                      add=True)
                     for t in range(4)]
        for bb in (0, 1):
            for d in sd[bb]:
                d.wait()
        plsc.subcore_barrier()
        pltpu.sync_copy(acc.at[pl.ds(sid * RPS, RPS)],
                        out_hbm.at[pl.ds(cid * HALF + sid * RPS, RPS)])

    return sk(vals, idx3, zeros_half)


def kernel(edge_index, rels, dists, query_rels, edge_conf_mask, edge_mask,
           node_mask, scores, conf_B, conf_W, conf_b, rel_table, lre_beta_W,
           lre_beta_b, lre_msg_W, lre_msg_b, lre_upd_W, lre_upd_b, lre_ln_g,
           lre_ln_b, dist_table, sfe_msg_W, sfe_msg_b, sfe_upd_W, sfe_upd_b,
           att_W, att_b, Wq, bq, Wk, bk, Wv, bv, fmr_ln_g, fmr_ln_b):
    f32 = jnp.float32
    src = edge_index[:, 0, :].astype(jnp.int32).reshape(BE)
    dst = edge_index[:, 1, :].astype(jnp.int32).reshape(BE)
    boff = jnp.repeat(jnp.arange(B_, dtype=jnp.int32) * N_, E_)
    srcg = src + boff
    dstg = dst + boff
    rels_f = rels.astype(jnp.int32).reshape(BE)
    dclip = jnp.clip(dists, 0, 9).astype(jnp.int32).reshape(BN)
    scores_f = scores.astype(f32).reshape(BE, 1)
    ecm_f = edge_conf_mask.astype(f32).reshape(BE, 1)
    src0 = (src == 0).astype(f32).reshape(BE, 1)
    qr2 = query_rels.astype(jnp.int32).reshape(B_, 1)
    conf_b2 = conf_b.reshape(1, D_)
    beta_b2 = lre_beta_b.reshape(1, 1)
    zeros_half = jnp.zeros((BN // NC, D_), f32)
    idx3 = dstg.reshape(NW, BE // NW // 128, 128)
    idx3 = idx3 - (jnp.arange(NW, dtype=jnp.int32)[:, None, None] // NS) * (BN // NC)

    dclip16 = jnp.broadcast_to(dclip.astype(f32)[:, None], (BN, 16))
    h_r, dist_emb, d16 = _gather_consts(rel_table, rels_f, dist_table,
                                        dclip, dclip16, srcg)

    conf, gate = _precompute(scores_f, ecm_f, h_r, qr2, conf_B, conf_W,
                             conf_b2, rel_table, lre_beta_W, beta_b2)

    lre_g = lre_ln_g.reshape(1, D_)
    lre_b = lre_ln_b.reshape(1, D_)
    h = jnp.zeros((BN, D_), f32).at[jnp.arange(B_) * N_].set(1.0)
    noise = jax.random.normal(jax.random.key(42), (B_, N_, D_)).reshape(BN, D_) * 0.1
    h2 = _add2(dist_emb, noise.astype(f32))
    hs_list = []
    for k in range(NL):
        Wk_full = lre_msg_W[k]
        Wc = jnp.concatenate([Wk_full[0:D_], Wk_full[D_:2 * D_],
                              Wk_full[3 * D_:4 * D_], Wk_full[4 * D_:5 * D_]], axis=0)
        csum3 = jnp.sum(Wk_full[2 * D_:3 * D_], axis=0).reshape(1, D_)
        bk_row = lre_msg_b[k].reshape(1, D_)
        W2_full = sfe_msg_W[k]
        Wc2 = jnp.concatenate([W2_full[0:D_], W2_full[D_:2 * D_],
                               W2_full[3 * D_:4 * D_], W2_full[4 * D_:5 * D_]], axis=0)
        tbl10 = dist_table @ W2_full[2 * D_:3 * D_]
        h_src = _gather_rows(h, srcg)
        h2_src = _gather_rows(h2, srcg)
        wm = _lre_msg(h_src, h_r, conf, gate, src0, Wc, csum3, bk_row)
        wm2 = _sfe_msg(h2_src, h_r, d16, conf, Wc2, tbl10,
                       sfe_msg_b[k].reshape(1, D_))
        aggr = _scatter_add_bn(wm, idx3, zeros_half)
        aggr2 = _scatter_add_bn(wm2, idx3, zeros_half)
        h = _lre_upd(aggr, h, lre_upd_W[k],
                     lre_upd_b[k].reshape(1, D_), lre_g, lre_b)
        h2 = _sfe_upd(aggr2, h2, sfe_upd_W[k],
                      sfe_upd_b[k].reshape(1, D_))
        hs_list.append(h)

    ctx_all = jnp.stack(
        [hk.reshape(B_, N_, D_)[:, 0, :] for hk in hs_list], axis=1)
    ctx_all = jnp.concatenate(
        [ctx_all, jnp.zeros((B_, 8 - NL, D_), f32)], axis=1)
    return _finale(ctx_all, h2, qr2, rel_table,
                   att_W[0:D_], att_W[D_:2 * D_], att_b.reshape(1, 1),
                   Wq, bq.reshape(1, D_), Wk, bk.reshape(1, D_),
                   Wv, bv.reshape(1, D_), fmr_ln_g.reshape(1, D_),
                   fmr_ln_b.reshape(1, D_))


# speedup vs baseline: 1.0323x; 1.0323x over previous
import functools
import math

import jax
import jax.numpy as jnp
from jax import lax
from jax.experimental import pallas as pl
from jax.experimental.pallas import tpu as pltpu

B_, N_, E_, D_ = 4, 2048, 16384, 64
NR, NL, TAU, M_ = 500, 3, 0.1, 20
BE = B_ * E_
BN = B_ * N_
EC = 2048
NEC = BE // EC
CPB = E_ // EC


def _pre_body(scores_ref, ecm_ref, hr_ref, qr_ref, confB_ref, confW_ref,
              confb_ref, rel_ref, betaW_ref, betab_ref, conf_ref, gate_ref):
    b = pl.program_id(0) // CPB
    s = scores_ref[...]
    m = ecm_ref[...]
    s3 = s * m
    xp = (2.0 * math.pi) * s3 * confB_ref[...]
    cs = jnp.concatenate([jnp.cos(xp), jnp.sin(xp)], axis=1)
    conf_ref[...] = cs @ confW_ref[...] + confb_ref[...]
    rtb = rel_ref[...] @ betaW_ref[...]
    qr = qr_ref[...]
    i500 = lax.broadcasted_iota(jnp.int32, (B_, NR), 1)
    qoh = (qr == i500).astype(jnp.float32)
    rqbw = qoh @ rtb
    i4 = lax.broadcasted_iota(jnp.int32, (B_, 1), 0)
    rqbw_b = jnp.sum(jnp.where(i4 == b, rqbw, 0.0), axis=0, keepdims=True)
    beta = jax.nn.sigmoid(hr_ref[...] @ betaW_ref[...] + rqbw_b + betab_ref[...])
    gate = m * jax.nn.sigmoid((s - beta) / TAU) + (1.0 - m) * 0.5
    gate_ref[...] = gate


def _precompute(scores_f, ecm_f, h_r, query_rels, conf_B, conf_W, conf_b,
                rel_table, beta_W, beta_b):
    full = lambda shape: pl.BlockSpec(shape, lambda i: (0, 0))
    chunk = lambda w: pl.BlockSpec((EC, w), lambda i: (i, 0))
    return pl.pallas_call(
        _pre_body,
        grid=(NEC,),
        in_specs=[chunk(1), chunk(1), chunk(D_), full((B_, 1)),
                  full((1, D_ // 2)), full((D_, D_)), full((1, D_)),
                  full((NR, D_)), full((D_, 1)), full((1, 1))],
        out_specs=[chunk(D_), chunk(1)],
        out_shape=[jax.ShapeDtypeStruct((BE, D_), jnp.float32),
                   jax.ShapeDtypeStruct((BE, 1), jnp.float32)],
    )(scores_f, ecm_f, h_r, query_rels, conf_B, conf_W, conf_b,
      rel_table, beta_W, beta_b)


def _msg_both_body(hs_ref, h2s_ref, hr_ref, cf_ref, gate_ref, src0_ref,
                   d16_ref, Wc_ref, csum_ref, bk_ref, Wc2_ref, t10_ref,
                   bk2_ref, wm_ref, wm2_ref):
    hs = hs_ref[...]
    h2s = h2s_ref[...]
    hr = hr_ref[...]
    cf = cf_ref[...]
    x = jnp.concatenate([hs * hr, hs, hr, cf], axis=1)
    raw = x @ Wc_ref[...] + src0_ref[...] * csum_ref[...] + bk_ref[...]
    wm_ref[...] = gate_ref[...] * jnp.maximum(raw, 0.0)
    x2 = jnp.concatenate([h2s * hr, h2s, hr, cf], axis=1)
    dval = d16_ref[...][:, 0:1]
    i10 = lax.broadcasted_iota(jnp.int32, (EC, 10), 1).astype(jnp.float32)
    oneh = (dval == i10).astype(jnp.float32)
    wm2_ref[...] = jnp.maximum(
        x2 @ Wc2_ref[...] + oneh @ t10_ref[...] + bk2_ref[...], 0.0)


def _msg_both(h_src, h2_src, h_r, conf, gate, src0, d16,
              Wc, csum3, bk, Wc2, tbl10, bk2):
    full = lambda shape: pl.BlockSpec(shape, lambda i: (0, 0))
    chunk = lambda w: pl.BlockSpec((EC, w), lambda i: (i, 0))
    return pl.pallas_call(
        _msg_both_body,
        grid=(NEC,),
        in_specs=[chunk(D_), chunk(D_), chunk(D_), chunk(D_), chunk(1),
                  chunk(1), chunk(16),
                  full((4 * D_, D_)), full((1, D_)), full((1, D_)),
                  full((4 * D_, D_)), full((10, D_)), full((1, D_))],
        out_specs=[chunk(D_), chunk(D_)],
        out_shape=[jax.ShapeDtypeStruct((BE, D_), jnp.float32),
                   jax.ShapeDtypeStruct((BE, D_), jnp.float32)],
    )(h_src, h2_src, h_r, conf, gate, src0, d16, Wc, csum3, bk, Wc2,
      tbl10, bk2)


def _ln_rows(x, g, b):
    m = jnp.mean(x, axis=1, keepdims=True)
    v = jnp.mean((x - m) ** 2, axis=1, keepdims=True)
    return (x - m) / jnp.sqrt(v + 1e-5) * g + b


def _upd_both_body(p1_ref, p2_ref, h_ref, h2_ref, W1_ref, b1_ref, g_ref,
                   lb_ref, W2_ref, b2_ref, o1_ref, o2_ref):
    o1_ref[...] = _ln_rows(h_ref[...] + p1_ref[...] @ W1_ref[...] + b1_ref[...],
                           g_ref[...], lb_ref[...])
    o2_ref[...] = h2_ref[...] + p2_ref[...] @ W2_ref[...] + b2_ref[...]


def _upd_both(p1, p2, h, h2, W1, b1, g, lb, W2, b2):
    full = lambda shape: pl.BlockSpec(shape, lambda: (0, 0))
    return pl.pallas_call(
        _upd_both_body,
        in_specs=[full((BN, D_)), full((BN, D_)), full((BN, D_)),
                  full((BN, D_)), full((D_, D_)), full((1, D_)),
                  full((1, D_)), full((1, D_)), full((D_, D_)), full((1, D_))],
        out_specs=[full((BN, D_)), full((BN, D_))],
        out_shape=[jax.ShapeDtypeStruct((BN, D_), jnp.float32),
                   jax.ShapeDtypeStruct((BN, D_), jnp.float32)],
    )(p1, p2, h, h2, W1, b1, g, lb, W2, b2)


def _add_body(a_ref, b_ref, o_ref):
    o_ref[...] = a_ref[...] + b_ref[...]


def _add2(a, b):
    full = pl.BlockSpec((BN, D_), lambda: (0, 0))
    return pl.pallas_call(
        _add_body,
        in_specs=[full, full],
        out_specs=full,
        out_shape=jax.ShapeDtypeStruct((BN, D_), jnp.float32),
    )(a, b)


def _finale_body(ctx_ref, h2f_ref, qr_ref, rel_ref,
                 attW1_ref, attW2_ref, attb_ref, Wq_ref, bq_ref, Wk_ref,
                 bk_ref, Wv_ref, bv_ref, g_ref, lb_ref, o_ref):
    b = pl.program_id(0)
    h2f = h2f_ref[...]
    t_state = h2f[0:1, :]
    qr = qr_ref[...]
    i4 = lax.broadcasted_iota(jnp.int32, (B_, 1), 0)
    qr_b = jnp.sum(jnp.where(i4 == b, qr, 0), axis=0, keepdims=True)
    i500c = lax.broadcasted_iota(jnp.int32, (1, NR), 1)
    qoh = (i500c == qr_b).astype(jnp.float32)
    rq = qoh @ rel_ref[...]
    att = h2f @ attW1_ref[...] + (rq @ attW2_ref[...] + attb_ref[...])
    att = jnp.where(att >= 0.0, att, 0.01 * att)
    amax = jnp.max(att, axis=0, keepdims=True)
    ex = jnp.exp(att - amax)
    alpha = ex / jnp.sum(ex, axis=0, keepdims=True)
    iota = lax.broadcasted_iota(jnp.int32, (N_, 1), 0)
    acur = alpha
    rows = []
    for _ in range(M_):
        v = jnp.max(acur, axis=0, keepdims=True)
        eq = acur == v
        fidx = jnp.min(jnp.where(eq, iota, N_), axis=0, keepdims=True)
        sel = iota == fidx
        ohf = sel.astype(jnp.float32)
        rows.append(jnp.sum(ohf * h2f, axis=0, keepdims=True) * v)
        acur = jnp.where(sel, -1.0, acur)
    ctx = ctx_ref[...].reshape(8, D_)[0:NL, :]
    x = jnp.concatenate([ctx] + rows, axis=0)
    Nt = NL + M_
    hd = lax.broadcasted_iota(jnp.int32, (D_, D_), 0) // 16
    hD = lax.broadcasted_iota(jnp.int32, (D_, D_), 1) // 16
    blockones = (hd == hD).astype(jnp.float32)
    q = x @ Wq_ref[...] + bq_ref[...]
    k_ = x @ Wk_ref[...] + bk_ref[...]
    v_ = x @ Wv_ref[...] + bv_ref[...]

    def nrmh(t):
        ssum = (t * t) @ blockones
        return t / jnp.maximum(jnp.sqrt(ssum), 1e-12)

    q = nrmh(q)
    k_ = nrmh(k_)
    KtV = lax.dot_general(k_, v_, (((0,), (0,)), ((), ())))
    kvs = KtV * blockones
    vsum = jnp.sum(v_, axis=0, keepdims=True)
    ksum = jnp.sum(k_, axis=0, keepdims=True)
    num = q @ kvs + vsum + v_ * float(Nt)
    den = (q * ksum) @ blockones + float(2 * Nt)
    out = num / den
    y = _ln_rows(x + out, g_ref[...], lb_ref[...])
    res = jnp.mean(y, axis=0, keepdims=True) + t_state
    o_ref[...] = jnp.concatenate(
        [res, jnp.zeros((7, D_), jnp.float32)], axis=0).reshape(1, 8, D_)


def _finale(ctx_all, h2f, query_rels, rel_table, attW1, attW2, attb,
            Wq, bq, Wk, bk, Wv, bv, g, lb):
    full = lambda shape: pl.BlockSpec(shape, lambda b: tuple(0 for _ in shape))
    return pl.pallas_call(
        _finale_body,
        grid=(B_,),
        in_specs=[pl.BlockSpec((1, 8, D_), lambda b: (b, 0, 0)),
                  pl.BlockSpec((N_, D_), lambda b: (b, 0)),
                  full((B_, 1)), full((NR, D_)),
                  full((D_, 1)), full((D_, 1)), full((1, 1)),
                  full((D_, D_)), full((1, D_)), full((D_, D_)), full((1, D_)),
                  full((D_, D_)), full((1, D_)), full((1, D_)), full((1, D_))],
        out_specs=pl.BlockSpec((1, 8, D_), lambda b: (b, 0, 0)),
        out_shape=jax.ShapeDtypeStruct((B_, 8, D_), jnp.float32),
    )(ctx_all, h2f, query_rels, rel_table, attW1, attW2, attb,
      Wq, bq, Wk, bk, Wv, bv, g, lb)[:, 0, :]


NC, NS = 2, 16
NW = NC * NS


def _sc_mesh():
    from jax.experimental.pallas import tpu_sc as plsc
    return plsc.VectorSubcoreMesh(core_axis_name="c", subcore_axis_name="s")


def _gather2(tab1, tab2, idx):
    per_w = BE // NW
    c = 256
    nch = per_w // c

    @functools.partial(
        pl.kernel,
        out_type=(jax.ShapeDtypeStruct((BE, D_), jnp.float32),
                  jax.ShapeDtypeStruct((BE, D_), jnp.float32)),
        mesh=_sc_mesh(),
        scratch_types=[pltpu.VMEM((per_w,), jnp.int32),
                       pltpu.VMEM((2, 2, c, D_), jnp.float32),
                       pltpu.SemaphoreType.DMA, pltpu.SemaphoreType.DMA,
                       pltpu.SemaphoreType.DMA, pltpu.SemaphoreType.DMA],
        compiler_params=pltpu.CompilerParams(use_tc_tiling_on_sc=False),
    )
    def gk(t1_hbm, t2_hbm, idx_hbm, o1_hbm, o2_hbm, idx_v, rows_v,
           g0, g1, w0, w1):
        w = lax.axis_index("c") * NS + lax.axis_index("s")
        base = w * per_w
        pltpu.sync_copy(idx_hbm.at[pl.ds(base, per_w)], idx_v)
        gsem = [g0, g1]
        wsem = [w0, w1]
        gd = [[], []]
        wd = [[], []]

        def fire(j, b):
            s = idx_v.at[pl.ds(j * c, c)]
            gd[b] = [pltpu.async_copy(t1_hbm.at[s], rows_v.at[b, 0], gsem[b]),
                     pltpu.async_copy(t2_hbm.at[s], rows_v.at[b, 1], gsem[b])]

        fire(0, 0)
        for j in range(nch):
            b = j % 2
            nb = (j + 1) % 2
            if j + 1 < nch:
                for d in wd[nb]:
                    d.wait()
                fire(j + 1, nb)
            for d in gd[b]:
                d.wait()
            o = pl.ds(base + j * c, c)
            wd[b] = [pltpu.async_copy(rows_v.at[b, 0], o1_hbm.at[o], wsem[b]),
                     pltpu.async_copy(rows_v.at[b, 1], o2_hbm.at[o], wsem[b])]
        for bb in (0, 1):
            for d in wd[bb]:
                d.wait()

    return gk(tab1, tab2, idx)


def _gather_consts(rel_table, rels_f, dist_table, dclip, dclip16, srcg):
    pwE = BE // NW
    pwN = BN // NW
    c = 512
    nch = pwE // c

    @functools.partial(
        pl.kernel,
        out_type=(jax.ShapeDtypeStruct((BE, D_), jnp.float32),
                  jax.ShapeDtypeStruct((BN, D_), jnp.float32),
                  jax.ShapeDtypeStruct((BE, 16), jnp.float32)),
        mesh=_sc_mesh(),
        scratch_types=[pltpu.VMEM((pwE,), jnp.int32),
                       pltpu.VMEM((pwN,), jnp.int32),
                       pltpu.VMEM((2, c, D_), jnp.float32),
                       pltpu.VMEM((pwN, D_), jnp.float32),
                       pltpu.VMEM((pwE, 16), jnp.float32),
                       pltpu.SemaphoreType.DMA, pltpu.SemaphoreType.DMA,
                       pltpu.SemaphoreType.DMA, pltpu.SemaphoreType.DMA,
                       pltpu.SemaphoreType.DMA, pltpu.SemaphoreType.DMA],
        compiler_params=pltpu.CompilerParams(use_tc_tiling_on_sc=False),
    )
    def gk(rel_hbm, rels_hbm, dtbl_hbm, dclip_hbm, d16tbl_hbm, srcg_hbm,
           hr_out, demb_out, d16_out, idxE, idxN, rows, rowsN, rows16,
           g0, g1, w0, w1, aux, aux2):
        w = lax.axis_index("c") * NS + lax.axis_index("s")
        baseE = w * pwE
        baseN = w * pwN
        pltpu.sync_copy(srcg_hbm.at[pl.ds(baseE, pwE)], idxE)
        d16g = pltpu.async_copy(d16tbl_hbm.at[idxE], rows16, aux2)
        pltpu.sync_copy(dclip_hbm.at[pl.ds(baseN, pwN)], idxN)
        dg = pltpu.async_copy(dtbl_hbm.at[idxN], rowsN, aux)
        d16g.wait()
        d16w = pltpu.async_copy(rows16, d16_out.at[pl.ds(baseE, pwE)], aux2)
        dg.wait()
        dw = pltpu.async_copy(rowsN, demb_out.at[pl.ds(baseN, pwN)], aux)
        pltpu.sync_copy(rels_hbm.at[pl.ds(baseE, pwE)], idxE)
        gsem = [g0, g1]
        wsem = [w0, w1]
        gd = [None, None]
        wd = [None, None]
        gd[0] = pltpu.async_copy(rel_hbm.at[idxE.at[pl.ds(0, c)]],
                                 rows.at[0], gsem[0])
        for j in range(nch):
            b = j % 2
            nb = (j + 1) % 2
            if j + 1 < nch:
                if wd[nb] is not None:
                    wd[nb].wait()
                gd[nb] = pltpu.async_copy(
                    rel_hbm.at[idxE.at[pl.ds((j + 1) * c, c)]],
                    rows.at[nb], gsem[nb])
            gd[b].wait()
            wd[b] = pltpu.async_copy(rows.at[b],
                                     hr_out.at[pl.ds(baseE + j * c, c)],
                                     wsem[b])
        for d in wd:
            if d is not None:
                d.wait()
        d16w.wait()
        dw.wait()

    return gk(rel_table, rels_f, dist_table, dclip, dclip16, srcg)


def _scatter2(vals1, vals2, idx3, zeros_half):
    HALF = BN // NC
    RPS = HALF // NS
    c = 256
    nch = (BE // NW) // c
    rpc = c // 128

    @functools.partial(
        pl.kernel,
        out_type=(jax.ShapeDtypeStruct((BN, D_), jnp.float32),
                  jax.ShapeDtypeStruct((BN, D_), jnp.float32)),
        mesh=_sc_mesh(),
        scratch_types=[pltpu.VMEM((BE // NW // 128, 128), jnp.int32),
                       pltpu.VMEM((2, 2, c, D_), jnp.float32),
                       pltpu.VMEM_SHARED((HALF, D_), jnp.float32),
                       pltpu.VMEM_SHARED((HALF, D_), jnp.float32),
                       pltpu.SemaphoreType.DMA, pltpu.SemaphoreType.DMA,
                       pltpu.SemaphoreType.DMA, pltpu.SemaphoreType.DMA],
        compiler_params=pltpu.CompilerParams(use_tc_tiling_on_sc=False),
    )
    def sk(v1_hbm, v2_hbm, idx_hbm, zeros_hbm, o1_hbm, o2_hbm,
           idx_v, vals_v, acc1, acc2, l0, l1, s0, s1):
        from jax.experimental.pallas import tpu_sc as plsc
        cid = lax.axis_index("c")
        sid = lax.axis_index("s")
        w = cid * NS + sid
        zs = pl.ds(sid * RPS, RPS)
        pltpu.sync_copy(zeros_hbm.at[zs], acc1.at[zs])
        pltpu.sync_copy(zeros_hbm.at[zs], acc2.at[zs])
        pltpu.sync_copy(idx_hbm.at[w], idx_v)
        plsc.subcore_barrier()
        base = w * (BE // NW)
        lsem = [l0, l1]
        ssem = [s0, s1]
        ld = [[], []]
        sd = [[], []]

        def fire_load(j, b):
            s = pl.ds(base + j * c, c)
            ld[b] = [pltpu.async_copy(v1_hbm.at[s], vals_v.at[b, 0], lsem[b]),
                     pltpu.async_copy(v2_hbm.at[s], vals_v.at[b, 1], lsem[b])]

        fire_load(0, 0)
        for j in range(nch):
            b = j % 2
            nb = (j + 1) % 2
            if j + 1 < nch:
                for d in sd[nb]:
                    d.wait()
                sd[nb] = []
                fire_load(j + 1, nb)
            for d in ld[b]:
                d.wait()
            sd[b] = []
            for t in range(rpc):
                irow = idx_v.at[j * rpc + t]
                vs = pl.ds(t * 128, 128)
                sd[b].append(pltpu.async_copy(vals_v.at[b, 0].at[vs],
                                              acc1.at[irow], ssem[b], add=True))
                sd[b].append(pltpu.async_copy(vals_v.at[b, 1].at[vs],
                                              acc2.at[irow], ssem[b], add=True))
        for bb in (0, 1):
            for d in sd[bb]:
                d.wait()
        plsc.subcore_barrier()
        os = pl.ds(cid * HALF + sid * RPS, RPS)
        pltpu.sync_copy(acc1.at[zs], o1_hbm.at[os])
        pltpu.sync_copy(acc2.at[zs], o2_hbm.at[os])

    return sk(vals1, vals2, idx3, zeros_half)


def kernel(edge_index, rels, dists, query_rels, edge_conf_mask, edge_mask,
           node_mask, scores, conf_B, conf_W, conf_b, rel_table, lre_beta_W,
           lre_beta_b, lre_msg_W, lre_msg_b, lre_upd_W, lre_upd_b, lre_ln_g,
           lre_ln_b, dist_table, sfe_msg_W, sfe_msg_b, sfe_upd_W, sfe_upd_b,
           att_W, att_b, Wq, bq, Wk, bk, Wv, bv, fmr_ln_g, fmr_ln_b):
    f32 = jnp.float32
    src = edge_index[:, 0, :].astype(jnp.int32).reshape(BE)
    dst = edge_index[:, 1, :].astype(jnp.int32).reshape(BE)
    boff = jnp.repeat(jnp.arange(B_, dtype=jnp.int32) * N_, E_)
    srcg = src + boff
    dstg = dst + boff
    rels_f = rels.astype(jnp.int32).reshape(BE)
    dclip = jnp.clip(dists, 0, 9).astype(jnp.int32).reshape(BN)
    scores_f = scores.astype(f32).reshape(BE, 1)
    ecm_f = edge_conf_mask.astype(f32).reshape(BE, 1)
    src0 = (src == 0).astype(f32).reshape(BE, 1)
    qr2 = query_rels.astype(jnp.int32).reshape(B_, 1)
    conf_b2 = conf_b.reshape(1, D_)
    beta_b2 = lre_beta_b.reshape(1, 1)
    zeros_half = jnp.zeros((BN // NC, D_), f32)
    idx3 = dstg.reshape(NW, BE // NW // 128, 128)
    idx3 = idx3 - (jnp.arange(NW, dtype=jnp.int32)[:, None, None] // NS) * (BN // NC)

    dclip16 = jnp.broadcast_to(dclip.astype(f32)[:, None], (BN, 16))
    h_r, dist_emb, d16 = _gather_consts(rel_table, rels_f, dist_table,
                                        dclip, dclip16, srcg)

    conf, gate = _precompute(scores_f, ecm_f, h_r, qr2, conf_B, conf_W,
                             conf_b2, rel_table, lre_beta_W, beta_b2)

    lre_g = lre_ln_g.reshape(1, D_)
    lre_b = lre_ln_b.reshape(1, D_)
    h = jnp.zeros((BN, D_), f32).at[jnp.arange(B_) * N_].set(1.0)
    noise = jax.random.normal(jax.random.key(42), (B_, N_, D_)).reshape(BN, D_) * 0.1
    h2 = _add2(dist_emb, noise.astype(f32))
    hs_list = []
    for k in range(NL):
        Wk_full = lre_msg_W[k]
        Wc = jnp.concatenate([Wk_full[0:D_], Wk_full[D_:2 * D_],
                              Wk_full[3 * D_:4 * D_], Wk_full[4 * D_:5 * D_]], axis=0)
        csum3 = jnp.sum(Wk_full[2 * D_:3 * D_], axis=0).reshape(1, D_)
        bk_row = lre_msg_b[k].reshape(1, D_)
        W2_full = sfe_msg_W[k]
        Wc2 = jnp.concatenate([W2_full[0:D_], W2_full[D_:2 * D_],
                               W2_full[3 * D_:4 * D_], W2_full[4 * D_:5 * D_]], axis=0)
        tbl10 = dist_table @ W2_full[2 * D_:3 * D_]
        h_src, h2_src = _gather2(h, h2, srcg)
        wm, wm2 = _msg_both(h_src, h2_src, h_r, conf, gate, src0, d16,
                            Wc, csum3, bk_row, Wc2, tbl10,
                            sfe_msg_b[k].reshape(1, D_))
        aggr, aggr2 = _scatter2(wm, wm2, idx3, zeros_half)
        h, h2 = _upd_both(aggr, aggr2, h, h2, lre_upd_W[k],
                          lre_upd_b[k].reshape(1, D_), lre_g, lre_b,
                          sfe_upd_W[k], sfe_upd_b[k].reshape(1, D_))
        hs_list.append(h)

    ctx_all = jnp.stack(
        [hk.reshape(B_, N_, D_)[:, 0, :] for hk in hs_list], axis=1)
    ctx_all = jnp.concatenate(
        [ctx_all, jnp.zeros((B_, 8 - NL, D_), f32)], axis=1)
    return _finale(ctx_all, h2, qr2, rel_table,
                   att_W[0:D_], att_W[D_:2 * D_], att_b.reshape(1, 1),
                   Wq, bq.reshape(1, D_), Wk, bk.reshape(1, D_),
                   Wv, bv.reshape(1, D_), fmr_ln_g.reshape(1, D_),
                   fmr_ln_b.reshape(1, D_))


# speedup vs baseline: 1.6397x; 1.5883x over previous
import functools
import math

import jax
import jax.numpy as jnp
from jax import lax
from jax.experimental import pallas as pl
from jax.experimental.pallas import tpu as pltpu

B_, N_, E_, D_ = 4, 2048, 16384, 64
NR, NL, TAU, M_ = 500, 3, 0.1, 20
BE = B_ * E_
BN = B_ * N_
EC = 2048
NEC = BE // EC
CPB = E_ // EC


ECH = EC // 2
BEH = BE // 2
BNH = BN // 2


def _pre_body(scores_ref, ecm_ref, hr_ref, qr_ref, confB_ref, confW_ref,
              confb_ref, rel_ref, betaW_ref, betab_ref, conf_ref, gate_ref):
    b = pl.program_id(0) // CPB
    rtb = rel_ref[...] @ betaW_ref[...]
    qr = qr_ref[...]
    i500 = lax.broadcasted_iota(jnp.int32, (B_, NR), 1)
    qoh = (qr == i500).astype(jnp.float32)
    rqbw = qoh @ rtb
    i4 = lax.broadcasted_iota(jnp.int32, (B_, 1), 0)
    rqbw_b = jnp.sum(jnp.where(i4 == b, rqbw, 0.0), axis=0, keepdims=True)
    confs = []
    gates = []
    for hh in (0, 1):
        s = scores_ref[...][:, hh:hh + 1]
        m = ecm_ref[...][:, hh:hh + 1]
        hr = hr_ref[...][:, hh * D_:(hh + 1) * D_]
        s3 = s * m
        xp = (2.0 * math.pi) * s3 * confB_ref[...]
        cs = jnp.concatenate([jnp.cos(xp), jnp.sin(xp)], axis=1)
        confs.append(cs @ confW_ref[...] + confb_ref[...])
        beta = jax.nn.sigmoid(hr @ betaW_ref[...] + rqbw_b + betab_ref[...])
        gates.append(m * jax.nn.sigmoid((s - beta) / TAU) + (1.0 - m) * 0.5)
    conf_ref[...] = jnp.concatenate(confs, axis=1)
    gate_ref[...] = jnp.concatenate(gates, axis=1)


def _precompute(scores_f, ecm_f, h_r, query_rels, conf_B, conf_W, conf_b,
                rel_table, beta_W, beta_b):
    full = lambda shape: pl.BlockSpec(shape, lambda i: (0, 0))
    chunk = lambda w: pl.BlockSpec((ECH, w), lambda i: (i, 0))
    return pl.pallas_call(
        _pre_body,
        grid=(NEC,),
        in_specs=[chunk(2), chunk(2), chunk(2 * D_), full((B_, 1)),
                  full((1, D_ // 2)), full((D_, D_)), full((1, D_)),
                  full((NR, D_)), full((D_, 1)), full((1, 1))],
        out_specs=[chunk(2 * D_), chunk(2)],
        out_shape=[jax.ShapeDtypeStruct((BEH, 2 * D_), jnp.float32),
                   jax.ShapeDtypeStruct((BEH, 2), jnp.float32)],
    )(scores_f, ecm_f, h_r, query_rels, conf_B, conf_W, conf_b,
      rel_table, beta_W, beta_b)


def _msg_both_body(hs_ref, h2s_ref, hr_ref, cf_ref, gate_ref, src0_ref,
                   d16_ref, Wc_ref, csum_ref, bk_ref, Wc2_ref, t10_ref,
                   bk2_ref, wm_ref, wm2_ref):
    i10 = lax.broadcasted_iota(jnp.int32, (ECH, 10), 1).astype(jnp.float32)
    wms = []
    wm2s = []
    for hh in (0, 1):
        cs = slice(hh * D_, (hh + 1) * D_)
        hs = hs_ref[...][:, cs]
        h2s = h2s_ref[...][:, cs]
        hr = hr_ref[...][:, cs]
        cf = cf_ref[...][:, cs]
        gate = gate_ref[...][:, hh:hh + 1]
        src0 = src0_ref[...][:, hh:hh + 1]
        x = jnp.concatenate([hs * hr, hs, hr, cf], axis=1)
        raw = x @ Wc_ref[...] + src0 * csum_ref[...] + bk_ref[...]
        wms.append(gate * jnp.maximum(raw, 0.0))
        x2 = jnp.concatenate([h2s * hr, h2s, hr, cf], axis=1)
        dval = d16_ref[...][:, hh * 16:hh * 16 + 1]
        oneh = (dval == i10).astype(jnp.float32)
        wm2s.append(jnp.maximum(
            x2 @ Wc2_ref[...] + oneh @ t10_ref[...] + bk2_ref[...], 0.0))
    wm_ref[...] = jnp.concatenate(wms, axis=1)
    wm2_ref[...] = jnp.concatenate(wm2s, axis=1)


def _msg_both(h_src, h2_src, h_r, conf, gate, src0, d16,
              Wc, csum3, bk, Wc2, tbl10, bk2):
    full = lambda shape: pl.BlockSpec(shape, lambda i: (0, 0))
    chunk = lambda w: pl.BlockSpec((ECH, w), lambda i: (i, 0))
    return pl.pallas_call(
        _msg_both_body,
        grid=(NEC,),
        in_specs=[chunk(2 * D_), chunk(2 * D_), chunk(2 * D_), chunk(2 * D_),
                  chunk(2), chunk(2), chunk(32),
                  full((4 * D_, D_)), full((1, D_)), full((1, D_)),
                  full((4 * D_, D_)), full((10, D_)), full((1, D_))],
        out_specs=[chunk(2 * D_), chunk(2 * D_)],
        out_shape=[jax.ShapeDtypeStruct((BEH, 2 * D_), jnp.float32),
                   jax.ShapeDtypeStruct((BEH, 2 * D_), jnp.float32)],
    )(h_src, h2_src, h_r, conf, gate, src0, d16, Wc, csum3, bk, Wc2,
      tbl10, bk2)


def _ln_rows(x, g, b):
    m = jnp.mean(x, axis=1, keepdims=True)
    v = jnp.mean((x - m) ** 2, axis=1, keepdims=True)
    return (x - m) / jnp.sqrt(v + 1e-5) * g + b


def _upd_both_body(p1_ref, p2_ref, h_ref, h2_ref, W1_ref, b1_ref, g_ref,
                   lb_ref, W2_ref, b2_ref, o1_ref, o2_ref):
    o1s = []
    o2s = []
    for hh in (0, 1):
        cs = slice(hh * D_, (hh + 1) * D_)
        o1s.append(_ln_rows(
            h_ref[...][:, cs] + p1_ref[...][:, cs] @ W1_ref[...] + b1_ref[...],
            g_ref[...], lb_ref[...]))
        o2s.append(h2_ref[...][:, cs] + p2_ref[...][:, cs] @ W2_ref[...]
                   + b2_ref[...])
    o1_ref[...] = jnp.concatenate(o1s, axis=1)
    o2_ref[...] = jnp.concatenate(o2s, axis=1)


def _upd_both(p1, p2, h, h2, W1, b1, g, lb, W2, b2):
    full = lambda shape: pl.BlockSpec(shape, lambda: (0, 0))
    return pl.pallas_call(
        _upd_both_body,
        in_specs=[full((BNH, 2 * D_)), full((BNH, 2 * D_)),
                  full((BNH, 2 * D_)), full((BNH, 2 * D_)),
                  full((D_, D_)), full((1, D_)),
                  full((1, D_)), full((1, D_)), full((D_, D_)), full((1, D_))],
        out_specs=[full((BNH, 2 * D_)), full((BNH, 2 * D_))],
        out_shape=[jax.ShapeDtypeStruct((BNH, 2 * D_), jnp.float32),
                   jax.ShapeDtypeStruct((BNH, 2 * D_), jnp.float32)],
    )(p1, p2, h, h2, W1, b1, g, lb, W2, b2)


def _add_body(a_ref, b_ref, o_ref):
    o_ref[...] = a_ref[...] + b_ref[...]


def _add2(a, b):
    full = pl.BlockSpec((BNH, 2 * D_), lambda: (0, 0))
    return pl.pallas_call(
        _add_body,
        in_specs=[full, full],
        out_specs=full,
        out_shape=jax.ShapeDtypeStruct((BNH, 2 * D_), jnp.float32),
    )(a, b)


def _finale_body(ctx_ref, h2f_ref, qr_ref, rel_ref,
                 attW1_ref, attW2_ref, attb_ref, Wq_ref, bq_ref, Wk_ref,
                 bk_ref, Wv_ref, bv_ref, g_ref, lb_ref, o_ref):
    b = pl.program_id(0)
    h2f = h2f_ref[...]
    t_state = h2f[0:1, :]
    qr = qr_ref[...]
    i4 = lax.broadcasted_iota(jnp.int32, (B_, 1), 0)
    qr_b = jnp.sum(jnp.where(i4 == b, qr, 0), axis=0, keepdims=True)
    i500c = lax.broadcasted_iota(jnp.int32, (1, NR), 1)
    qoh = (i500c == qr_b).astype(jnp.float32)
    rq = qoh @ rel_ref[...]
    att = h2f @ attW1_ref[...] + (rq @ attW2_ref[...] + attb_ref[...])
    att = jnp.where(att >= 0.0, att, 0.01 * att)
    amax = jnp.max(att, axis=0, keepdims=True)
    ex = jnp.exp(att - amax)
    alpha = ex / jnp.sum(ex, axis=0, keepdims=True)
    iota = lax.broadcasted_iota(jnp.int32, (N_, 1), 0)
    acur = alpha
    rows = []
    for _ in range(M_):
        v = jnp.max(acur, axis=0, keepdims=True)
        eq = acur == v
        fidx = jnp.min(jnp.where(eq, iota, N_), axis=0, keepdims=True)
        sel = iota == fidx
        ohf = sel.astype(jnp.float32)
        rows.append(jnp.sum(ohf * h2f, axis=0, keepdims=True) * v)
        acur = jnp.where(sel, -1.0, acur)
    ctx = ctx_ref[...].reshape(8, D_)[0:NL, :]
    x = jnp.concatenate([ctx] + rows, axis=0)
    Nt = NL + M_
    hd = lax.broadcasted_iota(jnp.int32, (D_, D_), 0) // 16
    hD = lax.broadcasted_iota(jnp.int32, (D_, D_), 1) // 16
    blockones = (hd == hD).astype(jnp.float32)
    q = x @ Wq_ref[...] + bq_ref[...]
    k_ = x @ Wk_ref[...] + bk_ref[...]
    v_ = x @ Wv_ref[...] + bv_ref[...]

    def nrmh(t):
        ssum = (t * t) @ blockones
        return t / jnp.maximum(jnp.sqrt(ssum), 1e-12)

    q = nrmh(q)
    k_ = nrmh(k_)
    KtV = lax.dot_general(k_, v_, (((0,), (0,)), ((), ())))
    kvs = KtV * blockones
    vsum = jnp.sum(v_, axis=0, keepdims=True)
    ksum = jnp.sum(k_, axis=0, keepdims=True)
    num = q @ kvs + vsum + v_ * float(Nt)
    den = (q * ksum) @ blockones + float(2 * Nt)
    out = num / den
    y = _ln_rows(x + out, g_ref[...], lb_ref[...])
    res = jnp.mean(y, axis=0, keepdims=True) + t_state
    o_ref[...] = jnp.concatenate(
        [res, jnp.zeros((7, D_), jnp.float32)], axis=0).reshape(1, 8, D_)


def _finale(ctx_all, h2f, query_rels, rel_table, attW1, attW2, attb,
            Wq, bq, Wk, bk, Wv, bv, g, lb):
    full = lambda shape: pl.BlockSpec(shape, lambda b: tuple(0 for _ in shape))
    return pl.pallas_call(
        _finale_body,
        grid=(B_,),
        in_specs=[pl.BlockSpec((1, 8, D_), lambda b: (b, 0, 0)),
                  pl.BlockSpec((N_, D_), lambda b: (b, 0)),
                  full((B_, 1)), full((NR, D_)),
                  full((D_, 1)), full((D_, 1)), full((1, 1)),
                  full((D_, D_)), full((1, D_)), full((D_, D_)), full((1, D_)),
                  full((D_, D_)), full((1, D_)), full((1, D_)), full((1, D_))],
        out_specs=pl.BlockSpec((1, 8, D_), lambda b: (b, 0, 0)),
        out_shape=jax.ShapeDtypeStruct((B_, 8, D_), jnp.float32),
    )(ctx_all, h2f, query_rels, rel_table, attW1, attW2, attb,
      Wq, bq, Wk, bk, Wv, bv, g, lb)[:, 0, :]


NC, NS = 2, 16
NW = NC * NS


def _sc_mesh():
    from jax.experimental.pallas import tpu_sc as plsc
    return plsc.VectorSubcoreMesh(core_axis_name="c", subcore_axis_name="s")


def _gather2(tab1, tab2, idx):
    per_w = BE // NW
    c = 256
    nch = per_w // c

    @functools.partial(
        pl.kernel,
        out_type=(jax.ShapeDtypeStruct((BE, D_), jnp.float32),
                  jax.ShapeDtypeStruct((BE, D_), jnp.float32)),
        mesh=_sc_mesh(),
        scratch_types=[pltpu.VMEM((per_w,), jnp.int32),
                       pltpu.VMEM((2, 2, c, D_), jnp.float32),
                       pltpu.SemaphoreType.DMA, pltpu.SemaphoreType.DMA,
                       pltpu.SemaphoreType.DMA, pltpu.SemaphoreType.DMA],
        compiler_params=pltpu.CompilerParams(use_tc_tiling_on_sc=False),
    )
    def gk(t1_hbm, t2_hbm, idx_hbm, o1_hbm, o2_hbm, idx_v, rows_v,
           g0, g1, w0, w1):
        w = lax.axis_index("c") * NS + lax.axis_index("s")
        base = w * per_w
        pltpu.sync_copy(idx_hbm.at[pl.ds(base, per_w)], idx_v)
        gsem = [g0, g1]
        wsem = [w0, w1]
        gd = [[], []]
        wd = [[], []]

        def fire(j, b):
            s = idx_v.at[pl.ds(j * c, c)]
            gd[b] = [pltpu.async_copy(t1_hbm.at[s], rows_v.at[b, 0], gsem[b]),
                     pltpu.async_copy(t2_hbm.at[s], rows_v.at[b, 1], gsem[b])]

        fire(0, 0)
        for j in range(nch):
            b = j % 2
            nb = (j + 1) % 2
            if j + 1 < nch:
                for d in wd[nb]:
                    d.wait()
                fire(j + 1, nb)
            for d in gd[b]:
                d.wait()
            o = pl.ds(base + j * c, c)
            wd[b] = [pltpu.async_copy(rows_v.at[b, 0], o1_hbm.at[o], wsem[b]),
                     pltpu.async_copy(rows_v.at[b, 1], o2_hbm.at[o], wsem[b])]
        for bb in (0, 1):
            for d in wd[bb]:
                d.wait()

    return gk(tab1, tab2, idx)


def _gather_consts(rel_table, rels_f, dist_table, dclip, dclip16, srcg):
    pwE = BE // NW
    pwN = BN // NW
    c = 512
    nch = pwE // c

    @functools.partial(
        pl.kernel,
        out_type=(jax.ShapeDtypeStruct((BE, D_), jnp.float32),
                  jax.ShapeDtypeStruct((BN, D_), jnp.float32),
                  jax.ShapeDtypeStruct((BE, 16), jnp.float32)),
        mesh=_sc_mesh(),
        scratch_types=[pltpu.VMEM((pwE,), jnp.int32),
                       pltpu.VMEM((pwN,), jnp.int32),
                       pltpu.VMEM((2, c, D_), jnp.float32),
                       pltpu.VMEM((pwN, D_), jnp.float32),
                       pltpu.VMEM((pwE, 16), jnp.float32),
                       pltpu.SemaphoreType.DMA, pltpu.SemaphoreType.DMA,
                       pltpu.SemaphoreType.DMA, pltpu.SemaphoreType.DMA,
                       pltpu.SemaphoreType.DMA, pltpu.SemaphoreType.DMA],
        compiler_params=pltpu.CompilerParams(use_tc_tiling_on_sc=False),
    )
    def gk(rel_hbm, rels_hbm, dtbl_hbm, dclip_hbm, d16tbl_hbm, srcg_hbm,
           hr_out, demb_out, d16_out, idxE, idxN, rows, rowsN, rows16,
           g0, g1, w0, w1, aux, aux2):
        w = lax.axis_index("c") * NS + lax.axis_index("s")
        baseE = w * pwE
        baseN = w * pwN
        pltpu.sync_copy(srcg_hbm.at[pl.ds(baseE, pwE)], idxE)
        d16g = pltpu.async_copy(d16tbl_hbm.at[idxE], rows16, aux2)
        pltpu.sync_copy(dclip_hbm.at[pl.ds(baseN, pwN)], idxN)
        dg = pltpu.async_copy(dtbl_hbm.at[idxN], rowsN, aux)
        d16g.wait()
        d16w = pltpu.async_copy(rows16, d16_out.at[pl.ds(baseE, pwE)], aux2)
        dg.wait()
        dw = pltpu.async_copy(rowsN, demb_out.at[pl.ds(baseN, pwN)], aux)
        pltpu.sync_copy(rels_hbm.at[pl.ds(baseE, pwE)], idxE)
        gsem = [g0, g1]
        wsem = [w0, w1]
        gd = [None, None]
        wd = [None, None]
        gd[0] = pltpu.async_copy(rel_hbm.at[idxE.at[pl.ds(0, c)]],
                                 rows.at[0], gsem[0])
        for j in range(nch):
            b = j % 2
            nb = (j + 1) % 2
            if j + 1 < nch:
                if wd[nb] is not None:
                    wd[nb].wait()
                gd[nb] = pltpu.async_copy(
                    rel_hbm.at[idxE.at[pl.ds((j + 1) * c, c)]],
                    rows.at[nb], gsem[nb])
            gd[b].wait()
            wd[b] = pltpu.async_copy(rows.at[b],
                                     hr_out.at[pl.ds(baseE + j * c, c)],
                                     wsem[b])
        for d in wd:
            if d is not None:
                d.wait()
        d16w.wait()
        dw.wait()

    return gk(rel_table, rels_f, dist_table, dclip, dclip16, srcg)


def _scatter2(vals1, vals2, idx3, zeros_half):
    HALF = BN // NC
    RPS = HALF // NS
    c = 256
    nch = (BE // NW) // c
    rpc = c // 128

    @functools.partial(
        pl.kernel,
        out_type=(jax.ShapeDtypeStruct((BN, D_), jnp.float32),
                  jax.ShapeDtypeStruct((BN, D_), jnp.float32)),
        mesh=_sc_mesh(),
        scratch_types=[pltpu.VMEM((BE // NW // 128, 128), jnp.int32),
                       pltpu.VMEM((2, 2, c, D_), jnp.float32),
                       pltpu.VMEM_SHARED((HALF, D_), jnp.float32),
                       pltpu.VMEM_SHARED((HALF, D_), jnp.float32),
                       pltpu.SemaphoreType.DMA, pltpu.SemaphoreType.DMA,
                       pltpu.SemaphoreType.DMA, pltpu.SemaphoreType.DMA],
        compiler_params=pltpu.CompilerParams(use_tc_tiling_on_sc=False),
    )
    def sk(v1_hbm, v2_hbm, idx_hbm, zeros_hbm, o1_hbm, o2_hbm,
           idx_v, vals_v, acc1, acc2, l0, l1, s0, s1):
        from jax.experimental.pallas import tpu_sc as plsc
        cid = lax.axis_index("c")
        sid = lax.axis_index("s")
        w = cid * NS + sid
        zs = pl.ds(sid * RPS, RPS)
        pltpu.sync_copy(zeros_hbm.at[zs], acc1.at[zs])
        pltpu.sync_copy(zeros_hbm.at[zs], acc2.at[zs])
        pltpu.sync_copy(idx_hbm.at[w], idx_v)
        plsc.subcore_barrier()
        base = w * (BE // NW)
        lsem = [l0, l1]
        ssem = [s0, s1]
        ld = [[], []]
        sd = [[], []]

        def fire_load(j, b):
            s = pl.ds(base + j * c, c)
            ld[b] = [pltpu.async_copy(v1_hbm.at[s], vals_v.at[b, 0], lsem[b]),
                     pltpu.async_copy(v2_hbm.at[s], vals_v.at[b, 1], lsem[b])]

        fire_load(0, 0)
        for j in range(nch):
            b = j % 2
            nb = (j + 1) % 2
            if j + 1 < nch:
                for d in sd[nb]:
                    d.wait()
                sd[nb] = []
                fire_load(j + 1, nb)
            for d in ld[b]:
                d.wait()
            sd[b] = []
            for t in range(rpc):
                irow = idx_v.at[j * rpc + t]
                vs = pl.ds(t * 128, 128)
                sd[b].append(pltpu.async_copy(vals_v.at[b, 0].at[vs],
                                              acc1.at[irow], ssem[b], add=True))
                sd[b].append(pltpu.async_copy(vals_v.at[b, 1].at[vs],
                                              acc2.at[irow], ssem[b], add=True))
        for bb in (0, 1):
            for d in sd[bb]:
                d.wait()
        plsc.subcore_barrier()
        os = pl.ds(cid * HALF + sid * RPS, RPS)
        pltpu.sync_copy(acc1.at[zs], o1_hbm.at[os])
        pltpu.sync_copy(acc2.at[zs], o2_hbm.at[os])

    return sk(vals1, vals2, idx3, zeros_half)


def kernel(edge_index, rels, dists, query_rels, edge_conf_mask, edge_mask,
           node_mask, scores, conf_B, conf_W, conf_b, rel_table, lre_beta_W,
           lre_beta_b, lre_msg_W, lre_msg_b, lre_upd_W, lre_upd_b, lre_ln_g,
           lre_ln_b, dist_table, sfe_msg_W, sfe_msg_b, sfe_upd_W, sfe_upd_b,
           att_W, att_b, Wq, bq, Wk, bk, Wv, bv, fmr_ln_g, fmr_ln_b):
    f32 = jnp.float32
    src = edge_index[:, 0, :].astype(jnp.int32).reshape(BE)
    dst = edge_index[:, 1, :].astype(jnp.int32).reshape(BE)
    boff = jnp.repeat(jnp.arange(B_, dtype=jnp.int32) * N_, E_)
    srcg = src + boff
    dstg = dst + boff
    rels_f = rels.astype(jnp.int32).reshape(BE)
    dclip = jnp.clip(dists, 0, 9).astype(jnp.int32).reshape(BN)
    scores_f = scores.astype(f32).reshape(BEH, 2)
    ecm_f = edge_conf_mask.astype(f32).reshape(BEH, 2)
    src0 = (src == 0).astype(f32).reshape(BEH, 2)
    qr2 = query_rels.astype(jnp.int32).reshape(B_, 1)
    conf_b2 = conf_b.reshape(1, D_)
    beta_b2 = lre_beta_b.reshape(1, 1)
    zeros_half = jnp.zeros((BN // NC, D_), f32)
    idx3 = dstg.reshape(NW, BE // NW // 128, 128)
    idx3 = idx3 - (jnp.arange(NW, dtype=jnp.int32)[:, None, None] // NS) * (BN // NC)

    dclip16 = jnp.broadcast_to(dclip.astype(f32)[:, None], (BN, 16))
    h_r, dist_emb, d16 = _gather_consts(rel_table, rels_f, dist_table,
                                        dclip, dclip16, srcg)
    h_r = h_r.reshape(BEH, 2 * D_)
    d16 = d16.reshape(BEH, 32)

    conf, gate = _precompute(scores_f, ecm_f, h_r, qr2, conf_B, conf_W,
                             conf_b2, rel_table, lre_beta_W, beta_b2)

    lre_g = lre_ln_g.reshape(1, D_)
    lre_b = lre_ln_b.reshape(1, D_)
    ri = lax.broadcasted_iota(jnp.int32, (BNH, 2 * D_), 0)
    ci = lax.broadcasted_iota(jnp.int32, (BNH, 2 * D_), 1)
    h = jnp.where((ri % (N_ // 2) == 0) & (ci < D_), 1.0, 0.0).astype(f32)
    noise = jax.random.normal(jax.random.key(42), (B_, N_, D_)).reshape(BNH, 2 * D_) * 0.1
    h2 = _add2(dist_emb.reshape(BNH, 2 * D_), noise.astype(f32))
    hs_list = []
    for k in range(NL):
        Wk_full = lre_msg_W[k]
        Wc = jnp.concatenate([Wk_full[0:D_], Wk_full[D_:2 * D_],
                              Wk_full[3 * D_:4 * D_], Wk_full[4 * D_:5 * D_]], axis=0)
        csum3 = jnp.sum(Wk_full[2 * D_:3 * D_], axis=0).reshape(1, D_)
        bk_row = lre_msg_b[k].reshape(1, D_)
        W2_full = sfe_msg_W[k]
        Wc2 = jnp.concatenate([W2_full[0:D_], W2_full[D_:2 * D_],
                               W2_full[3 * D_:4 * D_], W2_full[4 * D_:5 * D_]], axis=0)
        tbl10 = dist_table @ W2_full[2 * D_:3 * D_]
        h_src, h2_src = _gather2(h.reshape(BN, D_), h2.reshape(BN, D_), srcg)
        wm, wm2 = _msg_both(h_src.reshape(BEH, 2 * D_),
                            h2_src.reshape(BEH, 2 * D_),
                            h_r, conf, gate, src0, d16,
                            Wc, csum3, bk_row, Wc2, tbl10,
                            sfe_msg_b[k].reshape(1, D_))
        aggr, aggr2 = _scatter2(wm.reshape(BE, D_), wm2.reshape(BE, D_),
                                idx3, zeros_half)
        h, h2 = _upd_both(aggr.reshape(BNH, 2 * D_), aggr2.reshape(BNH, 2 * D_),
                          h, h2, lre_upd_W[k],
                          lre_upd_b[k].reshape(1, D_), lre_g, lre_b,
                          sfe_upd_W[k], sfe_upd_b[k].reshape(1, D_))
        hs_list.append(h)

    ctx_all = jnp.stack(
        [hk.reshape(B_, N_ // 2, 2 * D_)[:, 0, :D_] for hk in hs_list],
        axis=1)
    ctx_all = jnp.concatenate(
        [ctx_all, jnp.zeros((B_, 8 - NL, D_), f32)], axis=1)
    h2 = h2.reshape(BN, D_)
    return _finale(ctx_all, h2, qr2, rel_table,
                   att_W[0:D_], att_W[D_:2 * D_], att_b.reshape(1, 1),
                   Wq, bq.reshape(1, D_), Wk, bk.reshape(1, D_),
                   Wv, bv.reshape(1, D_), fmr_ln_g.reshape(1, D_),
                   fmr_ln_b.reshape(1, D_))
